# Initial kernel scaffold; baseline (speedup 1.0000x reference)
import jax
import jax.numpy as jnp
from jax.experimental import pallas as pl


def kernel(edge_index, edge_type, node_indices, emb, w_bases1, comp1, root1, bias1, w_bases2, comp2, root2, bias2, ln1_g, ln1_b, ln2_g, ln2_b, cls_w1, cls_b1, cls_w2, cls_b2):
    raise NotImplementedError("write your pallas kernel here")


# speedup vs baseline: 15.5561x; 15.5561x over previous
import functools

import jax
import jax.numpy as jnp
from jax import lax
from jax.experimental import pallas as pl
from jax.experimental.pallas import tpu as pltpu
from jax.experimental.pallas import tpu_sc as plsc

N = 10000
H = 128
R = 8
E = 320000
Q = 2048
NCLS = 10
NB_BASES = 4

NCORES = 2
NSUB = 16
LANES = 16

K = R * N
_SC_PARAMS = pltpu.CompilerParams(use_tc_tiling_on_sc=False,
                                  needs_layout_passes=False)
KL = 16
CW = 80

EC = E // NSUB
NCH_A = EC // CW
NH = N // NCORES
KCH = 40
KTB = 312
KTR = 320

_mesh_a = plsc.VectorSubcoreMesh(core_axis_name="c", subcore_axis_name="s")


@functools.partial(
    pl.kernel,
    out_type=jax.ShapeDtypeStruct((N, KL), jnp.float32),
    mesh=_mesh_a,
    scratch_types=[
        pltpu.VMEM((EC,), jnp.int32),
        pltpu.VMEM((EC,), jnp.int32),
        pltpu.VMEM((NCH_A, CW), jnp.int32),
        pltpu.VMEM((NCH_A, CW), jnp.int32),
        pltpu.VMEM((CW, KL), jnp.float32),
        pltpu.VMEM((KCH, KL), jnp.float32),
        pltpu.VMEM((KCH, KL), jnp.float32),
        pltpu.VMEM_SHARED((NH + 8, KL), jnp.float32),
    ],
    compiler_params=_SC_PARAMS,
)
def _count_kernel(dst_hbm, typ_hbm, w_hbm,
                  dst_v, typ_v, dkey2, rkey2, vals, cbuf, wbuf, c_sh):
    cid = lax.axis_index("c")
    sid = lax.axis_index("s")
    one16 = jnp.ones((LANES,), jnp.float32)
    zero16 = jnp.zeros((LANES,), jnp.float32)
    iota16 = lax.iota(jnp.int32, LANES)

    def fill_zero_vals(i, _):
        vals[i, :] = zero16
        return 0
    lax.fori_loop(0, CW, fill_zero_vals, 0)

    def fill_zero(i, _):
        cbuf[i, :] = zero16
        return 0
    lax.fori_loop(0, KCH, fill_zero, 0)

    kb = sid * KTB
    for k in range(KTR // KCH):
        pltpu.sync_copy(cbuf, c_sh.at[pl.ds(kb + k * KCH, KCH)])

    @pl.when(sid == NSUB - 1)
    def _():
        pltpu.sync_copy(cbuf.at[pl.ds(0, 8)], c_sh.at[pl.ds(NH, 8)])
    plsc.subcore_barrier()

    eb = sid * EC
    pltpu.sync_copy(dst_hbm.at[pl.ds(eb, EC)], dst_v)
    pltpu.sync_copy(typ_hbm.at[pl.ds(eb, EC)], typ_v)

    noff = cid * NH

    def build(j, _):
        for v in range(CW // LANES):
            off = j * CW + v * LANES
            t16 = typ_v[pl.ds(off, LANES)]
            d16 = dst_v[pl.ds(off, LANES)]
            loc = d16 - noff
            inb = (loc >= 0) & (loc < NH)
            dkey2[j, pl.ds(v * LANES, LANES)] = jnp.where(inb, loc, NH)
            rkey2[j, pl.ds(v * LANES, LANES)] = t16
        return 0
    lax.fori_loop(0, NCH_A, build, 0)

    def scat(j, _):
        for v in range(CW // LANES):
            r16 = rkey2[j, pl.ds(v * LANES, LANES)]
            for t in range(LANES):
                rs = r16[t]
                vals[v * LANES + t, :] = jnp.where(iota16 == rs, one16,
                                                   zero16)
        pltpu.sync_copy(vals, c_sh.at[dkey2.at[j]], add=True)
        return 0
    lax.fori_loop(0, NCH_A, scat, 0)
    plsc.subcore_barrier()

    for k in range(KTR // KCH):
        pltpu.sync_copy(c_sh.at[pl.ds(kb + k * KCH, KCH)], cbuf)

        def wcalc(i, _):
            wbuf[i, :] = 1.0 / jnp.maximum(cbuf[i, :], 1.0)
            return 0
        lax.fori_loop(0, KCH, wcalc, 0)
        pltpu.sync_copy(wbuf, w_hbm.at[pl.ds(noff + kb + k * KCH, KCH)])


SCHUNK = 2000
NSC = EC // SCHUNK
NCH_S = SCHUNK // CW
ATB = 312
ATR = 320
ZR = 40

_mesh_c = plsc.VectorSubcoreMesh(core_axis_name="c", subcore_axis_name="s")


@functools.partial(
    pl.kernel,
    out_type=jax.ShapeDtypeStruct((N, H), jnp.float32),
    mesh=_mesh_c,
    scratch_types=[
        pltpu.VMEM((SCHUNK,), jnp.int32),
        pltpu.VMEM((SCHUNK,), jnp.int32),
        pltpu.VMEM((SCHUNK,), jnp.int32),
        pltpu.VMEM((NCH_S, CW), jnp.int32),
        pltpu.VMEM((NCH_S, CW), jnp.int32),
        pltpu.VMEM((NCH_S, CW), jnp.int32),
        pltpu.VMEM((NCH_S, CW), jnp.int32),
        pltpu.VMEM((CW, KL), jnp.float32),
        pltpu.VMEM((CW, H), jnp.float32),
        pltpu.VMEM((ZR, H), jnp.float32),
        pltpu.VMEM_SHARED((NH + 8, H), jnp.float32),
    ],
    compiler_params=_SC_PARAMS,
)
def _msg_kernel(xr_hbm, w_hbm, src_hbm, dst_hbm, typ_hbm, out_hbm,
                src_v, dst_v, typ_v, ksrc2, rkey2, dstg2, dstl2, wv, rows,
                zbuf, acc_sh):
    cid = lax.axis_index("c")
    sid = lax.axis_index("s")
    zero16 = jnp.zeros((LANES,), jnp.float32)
    iota16 = lax.iota(jnp.int32, LANES)
    noff = cid * NH

    def zb(i, _):
        for q in range(H // LANES):
            zbuf[i, pl.ds(q * LANES, LANES)] = zero16
        return 0
    lax.fori_loop(0, ZR, zb, 0)
    for k in range(ATR // ZR):
        pltpu.sync_copy(zbuf, acc_sh.at[pl.ds(sid * ATB + k * ZR, ZR)])

    @pl.when(sid == NSUB - 1)
    def _():
        pltpu.sync_copy(zbuf.at[pl.ds(0, 8)], acc_sh.at[pl.ds(NH, 8)])
    plsc.subcore_barrier()

    def super_chunk(sc, _):
        eb = pl.multiple_of(sid * EC + sc * SCHUNK, 8)
        pltpu.sync_copy(src_hbm.at[pl.ds(eb, SCHUNK)], src_v)
        pltpu.sync_copy(dst_hbm.at[pl.ds(eb, SCHUNK)], dst_v)
        pltpu.sync_copy(typ_hbm.at[pl.ds(eb, SCHUNK)], typ_v)

        def build(j, _):
            for v in range(CW // LANES):
                off = j * CW + v * LANES
                s16 = src_v[pl.ds(off, LANES)]
                d16 = dst_v[pl.ds(off, LANES)]
                t16 = typ_v[pl.ds(off, LANES)]
                loc = d16 - noff
                inb = (loc >= 0) & (loc < NH)
                sl = pl.ds(v * LANES, LANES)
                ksrc2[j, sl] = t16 * N + s16
                rkey2[j, sl] = t16
                dstg2[j, sl] = d16
                dstl2[j, sl] = jnp.where(inb, loc, NH)
            return 0
        lax.fori_loop(0, NCH_S, build, 0)

        def main(j, _):
            pltpu.sync_copy(xr_hbm.at[ksrc2.at[j]], rows)
            pltpu.sync_copy(w_hbm.at[dstg2.at[j]], wv)
            for v in range(CW // LANES):
                r16 = rkey2[j, pl.ds(v * LANES, LANES)]
                for t in range(LANES):
                    e = v * LANES + t
                    rs = r16[t]
                    ws = jnp.sum(jnp.where(iota16 == rs, wv[e, :], 0.0))
                    for q in range(H // LANES):
                        sl = pl.ds(q * LANES, LANES)
                        rows[e, sl] = rows[e, sl] * ws
            pltpu.sync_copy(rows, acc_sh.at[dstl2.at[j]], add=True)
            return 0
        lax.fori_loop(0, NCH_S, main, 0)
        return 0
    lax.fori_loop(0, NSC, super_chunk, 0)
    plsc.subcore_barrier()

    pltpu.sync_copy(acc_sh.at[pl.ds(sid * ATB, ATR)],
                    out_hbm.at[pl.ds(noff + sid * ATB, ATR)])


QT = Q // (NCORES * NSUB)


@functools.partial(
    pl.kernel,
    out_type=jax.ShapeDtypeStruct((Q, H), jnp.float32),
    mesh=_mesh_c,
    scratch_types=[
        pltpu.VMEM((QT,), jnp.int32),
        pltpu.VMEM((QT,), jnp.int32),
        pltpu.VMEM((QT, H), jnp.float32),
        pltpu.VMEM((QT, H), jnp.float32),
    ],
    compiler_params=_SC_PARAMS,
)
def _gather_kernel(xr_hbm, msg_hbm, nidx_hbm, out_hbm, i0, i1, b0, b1):
    cid = lax.axis_index("c")
    sid = lax.axis_index("s")
    gid = cid * NSUB + sid
    qb = gid * QT
    pltpu.sync_copy(nidx_hbm.at[pl.ds(qb, QT)], i1)

    def shift(v, _):
        sl = pl.ds(v * LANES, LANES)
        i0[sl] = i1[sl] + (R * N)
        return 0
    lax.fori_loop(0, QT // LANES, shift, 0)

    pltpu.sync_copy(xr_hbm.at[i0], b0)
    pltpu.sync_copy(msg_hbm.at[i1], b1)

    def add(i, _):
        for q in range(H // LANES):
            sl = pl.ds(q * LANES, LANES)
            b0[i, sl] = b0[i, sl] + b1[i, sl]
        return 0
    lax.fori_loop(0, QT, add, 0)
    pltpu.sync_copy(b0, out_hbm.at[pl.ds(qb, QT)])


BN = 1000
NBLK = N // BN


def _xr_block(h, comp_ref, bases_ref, root_ref, out_ref, r):
    @pl.when(r < R)
    def _():
        w = (comp_ref[r, 0] * bases_ref[0] + comp_ref[r, 1] * bases_ref[1]
             + comp_ref[r, 2] * bases_ref[2] + comp_ref[r, 3] * bases_ref[3])
        out_ref[0] = jnp.dot(h, w, preferred_element_type=jnp.float32)

    @pl.when(r == R)
    def _():
        out_ref[0] = jnp.dot(h, root_ref[...],
                             preferred_element_type=jnp.float32)


def _b1_body(comp_ref, x_ref, bases_ref, root_ref, out_ref):
    r = pl.program_id(1)
    _xr_block(x_ref[...], comp_ref, bases_ref, root_ref, out_ref, r)


def _b2_body(comp_ref, s_ref, p_ref, b1_ref, g_ref, bb_ref,
             bases_ref, root_ref, out_ref):
    r = pl.program_id(1)
    x = s_ref[0] + b1_ref[...] + p_ref[...]
    mu = jnp.mean(x, axis=-1, keepdims=True)
    var = jnp.mean((x - mu) ** 2, axis=-1, keepdims=True)
    x = (x - mu) * lax.rsqrt(var + 1e-5) * g_ref[...] + bb_ref[...]
    h = jnp.maximum(x, 0.0)
    _xr_block(h, comp_ref, bases_ref, root_ref, out_ref, r)


def _head_body(q_ref, b2_ref, g_ref, bb_ref, w1_ref, c1_ref, w2_ref, c2_ref,
               out_ref):
    x = q_ref[...] + b2_ref[...]
    mu = jnp.mean(x, axis=-1, keepdims=True)
    var = jnp.mean((x - mu) ** 2, axis=-1, keepdims=True)
    x = (x - mu) * lax.rsqrt(var + 1e-5) * g_ref[...] + bb_ref[...]
    h = jnp.maximum(
        jnp.dot(x, w1_ref[...], preferred_element_type=jnp.float32)
        + c1_ref[...], 0.0)
    out_ref[...] = (jnp.dot(h, w2_ref[...], preferred_element_type=jnp.float32)
                    + c2_ref[...])


_vec_spec = pl.BlockSpec((1, H), lambda nb, r: (0, 0))
_b1_call = pl.pallas_call(
    _b1_body,
    grid=(NBLK, R + 1),
    in_specs=[
        pl.BlockSpec(memory_space=pltpu.SMEM),
        pl.BlockSpec((BN, H), lambda nb, r: (nb, 0)),
        pl.BlockSpec((NB_BASES, H, H), lambda nb, r: (0, 0, 0)),
        pl.BlockSpec((H, H), lambda nb, r: (0, 0)),
    ],
    out_specs=pl.BlockSpec((1, BN, H), lambda nb, r: (r, nb, 0)),
    out_shape=jax.ShapeDtypeStruct((R + 1, N, H), jnp.float32),
)

_b2_call = pl.pallas_call(
    _b2_body,
    grid=(NBLK, R + 1),
    in_specs=[
        pl.BlockSpec(memory_space=pltpu.SMEM),
        pl.BlockSpec((1, BN, H), lambda nb, r: (R, nb, 0)),
        pl.BlockSpec((BN, H), lambda nb, r: (nb, 0)),
        _vec_spec,
        _vec_spec,
        _vec_spec,
        pl.BlockSpec((NB_BASES, H, H), lambda nb, r: (0, 0, 0)),
        pl.BlockSpec((H, H), lambda nb, r: (0, 0)),
    ],
    out_specs=pl.BlockSpec((1, BN, H), lambda nb, r: (r, nb, 0)),
    out_shape=jax.ShapeDtypeStruct((R + 1, N, H), jnp.float32),
)

_head_call = pl.pallas_call(
    _head_body,
    out_shape=jax.ShapeDtypeStruct((Q, H), jnp.float32),
)


def kernel(edge_index, edge_type, node_indices, emb, w_bases1, comp1, root1,
           bias1, w_bases2, comp2, root2, bias2, ln1_g, ln1_b, ln2_g, ln2_b,
           cls_w1, cls_b1, cls_w2, cls_b2):
    src = edge_index[0].astype(jnp.int32)
    dst = edge_index[1].astype(jnp.int32)
    typ = edge_type.astype(jnp.int32)
    nidx = node_indices.astype(jnp.int32)

    w_node = _count_kernel(dst, typ)
    xr1 = _b1_call(comp1, emb, w_bases1, root1)
    msg1 = _msg_kernel(xr1.reshape((R + 1) * N, H), w_node, src, dst, typ)
    xr2 = _b2_call(comp2, xr1, msg1,
                   bias1.reshape(1, H), ln1_g.reshape(1, H),
                   ln1_b.reshape(1, H), w_bases2, root2)
    xr2f = xr2.reshape((R + 1) * N, H)
    msg2 = _msg_kernel(xr2f, w_node, src, dst, typ)
    q = _gather_kernel(xr2f, msg2, nidx)

    w2p = jnp.pad(cls_w2, ((0, 0), (0, H - NCLS)))
    b2p = jnp.pad(cls_b2, (0, H - NCLS)).reshape(1, H)
    logits = _head_call(q, bias2.reshape(1, H), ln2_g.reshape(1, H),
                        ln2_b.reshape(1, H), cls_w1, cls_b1.reshape(1, H),
                        w2p, b2p)
    return logits[:, :NCLS]


# speedup vs baseline: 34.4765x; 2.2163x over previous
import functools

import jax
import jax.numpy as jnp
from jax import lax
from jax.experimental import pallas as pl
from jax.experimental.pallas import tpu as pltpu
from jax.experimental.pallas import tpu_sc as plsc

N = 10000
H = 128
R = 8
E = 320000
Q = 2048
NCLS = 10
NB_BASES = 4

NCORES = 2
NSUB = 16
LANES = 16

K = R * N
_SC_PARAMS = pltpu.CompilerParams(use_tc_tiling_on_sc=False,
                                  needs_layout_passes=False)
KL = 16
CW = 80

EC = E // NSUB
NCH_A = EC // CW
NH = N // NCORES
KCH = 40
KTB = 312
KTR = 320

_mesh_a = plsc.VectorSubcoreMesh(core_axis_name="c", subcore_axis_name="s")


@functools.partial(
    pl.kernel,
    out_type=jax.ShapeDtypeStruct((N, KL), jnp.float32),
    mesh=_mesh_a,
    scratch_types=[
        pltpu.VMEM((EC,), jnp.int32),
        pltpu.VMEM((EC,), jnp.int32),
        pltpu.VMEM((NCH_A, CW), jnp.int32),
        pltpu.VMEM((NCH_A, CW), jnp.int32),
        pltpu.VMEM((CW, KL), jnp.float32),
        pltpu.VMEM((KCH, KL), jnp.float32),
        pltpu.VMEM((KCH, KL), jnp.float32),
        pltpu.VMEM_SHARED((NH + 8, KL), jnp.float32),
    ],
    compiler_params=_SC_PARAMS,
)
def _count_kernel(dst_hbm, typ_hbm, w_hbm,
                  dst_v, typ_v, dkey2, rkey2, vals, cbuf, wbuf, c_sh):
    cid = lax.axis_index("c")
    sid = lax.axis_index("s")
    one16 = jnp.ones((LANES,), jnp.float32)
    zero16 = jnp.zeros((LANES,), jnp.float32)
    iota16 = lax.iota(jnp.int32, LANES)

    def fill_zero_vals(i, _):
        vals[i, :] = zero16
        return 0
    lax.fori_loop(0, CW, fill_zero_vals, 0)

    def fill_zero(i, _):
        cbuf[i, :] = zero16
        return 0
    lax.fori_loop(0, KCH, fill_zero, 0)

    kb = sid * KTB
    for k in range(KTR // KCH):
        pltpu.sync_copy(cbuf, c_sh.at[pl.ds(kb + k * KCH, KCH)])

    @pl.when(sid == NSUB - 1)
    def _():
        pltpu.sync_copy(cbuf.at[pl.ds(0, 8)], c_sh.at[pl.ds(NH, 8)])
    plsc.subcore_barrier()

    eb = sid * EC
    pltpu.sync_copy(dst_hbm.at[pl.ds(eb, EC)], dst_v)
    pltpu.sync_copy(typ_hbm.at[pl.ds(eb, EC)], typ_v)

    noff = cid * NH

    def build(j, _):
        for v in range(CW // LANES):
            off = j * CW + v * LANES
            t16 = typ_v[pl.ds(off, LANES)]
            d16 = dst_v[pl.ds(off, LANES)]
            loc = d16 - noff
            inb = (loc >= 0) & (loc < NH)
            dkey2[j, pl.ds(v * LANES, LANES)] = jnp.where(inb, loc, NH)
            rkey2[j, pl.ds(v * LANES, LANES)] = t16
        return 0
    lax.fori_loop(0, NCH_A, build, 0)

    def scat(j, _):
        for v in range(CW // LANES):
            r16 = rkey2[j, pl.ds(v * LANES, LANES)]
            for t in range(LANES):
                rs = r16[t]
                vals[v * LANES + t, :] = jnp.where(iota16 == rs, one16,
                                                   zero16)
        pltpu.sync_copy(vals, c_sh.at[dkey2.at[j]], add=True)
        return 0
    lax.fori_loop(0, NCH_A, scat, 0)
    plsc.subcore_barrier()

    for k in range(KTR // KCH):
        pltpu.sync_copy(c_sh.at[pl.ds(kb + k * KCH, KCH)], cbuf)

        def wcalc(i, _):
            wbuf[i, :] = 1.0 / jnp.maximum(cbuf[i, :], 1.0)
            return 0
        lax.fori_loop(0, KCH, wcalc, 0)
        pltpu.sync_copy(wbuf, w_hbm.at[pl.ds(noff + kb + k * KCH, KCH)])


ET = E // (NCORES * NSUB)
SCHUNK = 2000
NSC = ET // SCHUNK
NCH_S = SCHUNK // CW
NPAIR = NCH_S // 2
NT = 624
NT_TAIL = N - NSUB * NT
ZR = 48

_mesh_c = plsc.VectorSubcoreMesh(core_axis_name="c", subcore_axis_name="s")


@functools.partial(
    pl.kernel,
    out_type=jax.ShapeDtypeStruct((NCORES * N, H), jnp.float32),
    mesh=_mesh_c,
    scratch_types=[
        pltpu.VMEM((SCHUNK,), jnp.int32),
        pltpu.VMEM((SCHUNK,), jnp.int32),
        pltpu.VMEM((SCHUNK,), jnp.int32),
        pltpu.VMEM((NCH_S, CW), jnp.int32),
        pltpu.VMEM((NCH_S, CW), jnp.int32),
        pltpu.VMEM((NCH_S, CW), jnp.int32),
        pltpu.VMEM((CW, KL), jnp.float32),
        pltpu.VMEM((CW, KL), jnp.float32),
        pltpu.VMEM((CW, H), jnp.float32),
        pltpu.VMEM((CW, H), jnp.float32),
        pltpu.VMEM((ZR, H), jnp.float32),
        pltpu.SemaphoreType.DMA,
        pltpu.SemaphoreType.DMA,
        pltpu.SemaphoreType.DMA,
        pltpu.SemaphoreType.DMA,
        pltpu.VMEM_SHARED((N, H), jnp.float32),
    ],
    compiler_params=_SC_PARAMS,
)
def _msg_kernel(xr_hbm, w_hbm, src_hbm, dst_hbm, typ_hbm, out_hbm,
                src_v, dst_v, typ_v, ksrc2, rkey2, dst2,
                wv0, wv1, rows0, rows1, zbuf, gs0, gs1, ws0, ws1, acc_sh):
    cid = lax.axis_index("c")
    sid = lax.axis_index("s")
    gid = cid * NSUB + sid
    zero16 = jnp.zeros((LANES,), jnp.float32)
    iota16 = lax.iota(jnp.int32, LANES)

    def zb(i, _):
        for q in range(H // LANES):
            zbuf[i, pl.ds(q * LANES, LANES)] = zero16
        return 0
    lax.fori_loop(0, ZR, zb, 0)
    for k in range(NT // ZR):
        pltpu.sync_copy(zbuf, acc_sh.at[pl.ds(sid * NT + k * ZR, ZR)])

    @pl.when(sid == NSUB - 1)
    def _():
        pltpu.sync_copy(zbuf.at[pl.ds(0, NT_TAIL)],
                        acc_sh.at[pl.ds(NSUB * NT, NT_TAIL)])
    plsc.subcore_barrier()

    rbufs = (rows0, rows1)
    wbufs = (wv0, wv1)
    gsems = (gs0, gs1)
    wsems = (ws0, ws1)

    def issue(j, b):
        pltpu.async_copy(xr_hbm.at[ksrc2.at[j]], rbufs[b], gsems[b])
        pltpu.async_copy(w_hbm.at[dst2.at[j]], wbufs[b], wsems[b])

    def wait(j, b):
        pltpu.make_async_copy(xr_hbm.at[ksrc2.at[j]], rbufs[b],
                              gsems[b]).wait()
        pltpu.make_async_copy(w_hbm.at[dst2.at[j]], wbufs[b],
                              wsems[b]).wait()

    def process(j, b):
        rows, wv = rbufs[b], wbufs[b]
        for v in range(CW // LANES):
            r16 = rkey2[j, pl.ds(v * LANES, LANES)]
            for t in range(LANES):
                e = v * LANES + t
                rs = r16[t]
                ws = jnp.sum(jnp.where(iota16 == rs, wv[e, :], 0.0))
                for q in range(H // LANES):
                    sl = pl.ds(q * LANES, LANES)
                    rows[e, sl] = rows[e, sl] * ws
        pltpu.sync_copy(rows, acc_sh.at[dst2.at[j]], add=True)

    def super_chunk(sc, _):
        eb = pl.multiple_of(gid * ET + sc * SCHUNK, 8)
        pltpu.sync_copy(src_hbm.at[pl.ds(eb, SCHUNK)], src_v)
        pltpu.sync_copy(dst_hbm.at[pl.ds(eb, SCHUNK)], dst_v)
        pltpu.sync_copy(typ_hbm.at[pl.ds(eb, SCHUNK)], typ_v)

        def build(j, _):
            for v in range(CW // LANES):
                off = j * CW + v * LANES
                s16 = src_v[pl.ds(off, LANES)]
                d16 = dst_v[pl.ds(off, LANES)]
                t16 = typ_v[pl.ds(off, LANES)]
                sl = pl.ds(v * LANES, LANES)
                ksrc2[j, sl] = t16 * N + s16
                rkey2[j, sl] = t16
                dst2[j, sl] = d16
            return 0
        lax.fori_loop(0, NCH_S, build, 0)

        issue(0, 0)

        def pair(i, _):
            j0 = 2 * i
            wait(j0, 0)
            issue(j0 + 1, 1)
            process(j0, 0)
            wait(j0 + 1, 1)
            issue(j0 + 2, 0)
            process(j0 + 1, 1)
            return 0
        lax.fori_loop(0, NPAIR, pair, 0)
        wait(NCH_S - 1, 0)
        process(NCH_S - 1, 0)
        return 0
    lax.fori_loop(0, NSC, super_chunk, 0)
    plsc.subcore_barrier()

    pltpu.sync_copy(acc_sh.at[pl.ds(sid * NT, NT)],
                    out_hbm.at[pl.ds(cid * N + sid * NT, NT)])

    @pl.when(sid == NSUB - 1)
    def _():
        pltpu.sync_copy(acc_sh.at[pl.ds(NSUB * NT, NT_TAIL)],
                        out_hbm.at[pl.ds(cid * N + NSUB * NT, NT_TAIL)])


QT = Q // (NCORES * NSUB)


@functools.partial(
    pl.kernel,
    out_type=jax.ShapeDtypeStruct((Q, H), jnp.float32),
    mesh=_mesh_c,
    scratch_types=[
        pltpu.VMEM((QT,), jnp.int32),
        pltpu.VMEM((QT,), jnp.int32),
        pltpu.VMEM((QT,), jnp.int32),
        pltpu.VMEM((QT, H), jnp.float32),
        pltpu.VMEM((QT, H), jnp.float32),
        pltpu.VMEM((QT, H), jnp.float32),
    ],
    compiler_params=_SC_PARAMS,
)
def _gather_kernel(xr_hbm, parts_hbm, nidx_hbm, out_hbm,
                   i0, i1, i2, b0, b1, b2):
    cid = lax.axis_index("c")
    sid = lax.axis_index("s")
    gid = cid * NSUB + sid
    qb = gid * QT
    pltpu.sync_copy(nidx_hbm.at[pl.ds(qb, QT)], i1)

    def shift(v, _):
        sl = pl.ds(v * LANES, LANES)
        base = i1[sl]
        i0[sl] = base + (R * N)
        i2[sl] = base + N
        return 0
    lax.fori_loop(0, QT // LANES, shift, 0)

    pltpu.sync_copy(xr_hbm.at[i0], b0)
    pltpu.sync_copy(parts_hbm.at[i1], b1)
    pltpu.sync_copy(parts_hbm.at[i2], b2)

    def add(i, _):
        for q in range(H // LANES):
            sl = pl.ds(q * LANES, LANES)
            b0[i, sl] = b0[i, sl] + b1[i, sl] + b2[i, sl]
        return 0
    lax.fori_loop(0, QT, add, 0)
    pltpu.sync_copy(b0, out_hbm.at[pl.ds(qb, QT)])


BN = 1000
NBLK = N // BN


def _xr_block(h, comp_ref, bases_ref, root_ref, out_ref, r):
    @pl.when(r < R)
    def _():
        w = (comp_ref[r, 0] * bases_ref[0] + comp_ref[r, 1] * bases_ref[1]
             + comp_ref[r, 2] * bases_ref[2] + comp_ref[r, 3] * bases_ref[3])
        out_ref[0] = jnp.dot(h, w, preferred_element_type=jnp.float32)

    @pl.when(r == R)
    def _():
        out_ref[0] = jnp.dot(h, root_ref[...],
                             preferred_element_type=jnp.float32)


def _b1_body(comp_ref, x_ref, bases_ref, root_ref, out_ref):
    r = pl.program_id(1)
    _xr_block(x_ref[...], comp_ref, bases_ref, root_ref, out_ref, r)


def _b2_body(comp_ref, s_ref, p0_ref, p1_ref, b1_ref, g_ref, bb_ref,
             bases_ref, root_ref, out_ref):
    r = pl.program_id(1)
    x = s_ref[0] + b1_ref[...] + p0_ref[...] + p1_ref[...]
    mu = jnp.mean(x, axis=-1, keepdims=True)
    var = jnp.mean((x - mu) ** 2, axis=-1, keepdims=True)
    x = (x - mu) * lax.rsqrt(var + 1e-5) * g_ref[...] + bb_ref[...]
    h = jnp.maximum(x, 0.0)
    _xr_block(h, comp_ref, bases_ref, root_ref, out_ref, r)


def _head_body(q_ref, b2_ref, g_ref, bb_ref, w1_ref, c1_ref, w2_ref, c2_ref,
               out_ref):
    x = q_ref[...] + b2_ref[...]
    mu = jnp.mean(x, axis=-1, keepdims=True)
    var = jnp.mean((x - mu) ** 2, axis=-1, keepdims=True)
    x = (x - mu) * lax.rsqrt(var + 1e-5) * g_ref[...] + bb_ref[...]
    h = jnp.maximum(
        jnp.dot(x, w1_ref[...], preferred_element_type=jnp.float32)
        + c1_ref[...], 0.0)
    out_ref[...] = (jnp.dot(h, w2_ref[...], preferred_element_type=jnp.float32)
                    + c2_ref[...])


_vec_spec = pl.BlockSpec((1, H), lambda nb, r: (0, 0))
_b1_call = pl.pallas_call(
    _b1_body,
    grid=(NBLK, R + 1),
    in_specs=[
        pl.BlockSpec(memory_space=pltpu.SMEM),
        pl.BlockSpec((BN, H), lambda nb, r: (nb, 0)),
        pl.BlockSpec((NB_BASES, H, H), lambda nb, r: (0, 0, 0)),
        pl.BlockSpec((H, H), lambda nb, r: (0, 0)),
    ],
    out_specs=pl.BlockSpec((1, BN, H), lambda nb, r: (r, nb, 0)),
    out_shape=jax.ShapeDtypeStruct((R + 1, N, H), jnp.float32),
)

_b2_call = pl.pallas_call(
    _b2_body,
    grid=(NBLK, R + 1),
    in_specs=[
        pl.BlockSpec(memory_space=pltpu.SMEM),
        pl.BlockSpec((1, BN, H), lambda nb, r: (R, nb, 0)),
        pl.BlockSpec((BN, H), lambda nb, r: (nb, 0)),
        pl.BlockSpec((BN, H), lambda nb, r: (NBLK + nb, 0)),
        _vec_spec,
        _vec_spec,
        _vec_spec,
        pl.BlockSpec((NB_BASES, H, H), lambda nb, r: (0, 0, 0)),
        pl.BlockSpec((H, H), lambda nb, r: (0, 0)),
    ],
    out_specs=pl.BlockSpec((1, BN, H), lambda nb, r: (r, nb, 0)),
    out_shape=jax.ShapeDtypeStruct((R + 1, N, H), jnp.float32),
)

_head_call = pl.pallas_call(
    _head_body,
    out_shape=jax.ShapeDtypeStruct((Q, H), jnp.float32),
)


def kernel(edge_index, edge_type, node_indices, emb, w_bases1, comp1, root1,
           bias1, w_bases2, comp2, root2, bias2, ln1_g, ln1_b, ln2_g, ln2_b,
           cls_w1, cls_b1, cls_w2, cls_b2):
    src = edge_index[0].astype(jnp.int32)
    dst = edge_index[1].astype(jnp.int32)
    typ = edge_type.astype(jnp.int32)
    nidx = node_indices.astype(jnp.int32)

    w_node = _count_kernel(dst, typ)
    xr1 = _b1_call(comp1, emb, w_bases1, root1)
    parts1 = _msg_kernel(xr1.reshape((R + 1) * N, H), w_node, src, dst, typ)
    xr2 = _b2_call(comp2, xr1, parts1, parts1,
                   bias1.reshape(1, H), ln1_g.reshape(1, H),
                   ln1_b.reshape(1, H), w_bases2, root2)
    xr2f = xr2.reshape((R + 1) * N, H)
    parts2 = _msg_kernel(xr2f, w_node, src, dst, typ)
    q = _gather_kernel(xr2f, parts2, nidx)

    w2p = jnp.pad(cls_w2, ((0, 0), (0, H - NCLS)))
    b2p = jnp.pad(cls_b2, (0, H - NCLS)).reshape(1, H)
    logits = _head_call(q, bias2.reshape(1, H), ln2_g.reshape(1, H),
                        ln2_b.reshape(1, H), cls_w1, cls_b1.reshape(1, H),
                        w2p, b2p)
    return logits[:, :NCLS]


# speedup vs baseline: 49.4377x; 1.4340x over previous
import functools

import jax
import jax.numpy as jnp
from jax import lax
from jax.experimental import pallas as pl
from jax.experimental.pallas import tpu as pltpu
from jax.experimental.pallas import tpu_sc as plsc

N = 10000
H = 128
R = 8
E = 320000
Q = 2048
NCLS = 10
NB_BASES = 4

NCORES = 2
NSUB = 16
LANES = 16

K = R * N
_SC_PARAMS = pltpu.CompilerParams(use_tc_tiling_on_sc=False,
                                  needs_layout_passes=False)
KL = 16
CW = 80

ETA = E // (NCORES * NSUB)
ASC = 2000
ANSC = ETA // ASC
ANCH = ASC // CW
ANP = ANCH // 2
CTB = 624
CTT = N - NSUB * CTB
CZR = 48

_mesh_a = plsc.VectorSubcoreMesh(core_axis_name="c", subcore_axis_name="s")


@functools.partial(
    pl.kernel,
    out_type=jax.ShapeDtypeStruct((NCORES * N, KL), jnp.float32),
    mesh=_mesh_a,
    scratch_types=[
        pltpu.VMEM((ASC,), jnp.int32),
        pltpu.VMEM((ASC,), jnp.int32),
        pltpu.VMEM((ANCH, CW), jnp.int32),
        pltpu.VMEM((ANCH, CW), jnp.int32),
        pltpu.VMEM((CW, KL), jnp.float32),
        pltpu.VMEM((CW, KL), jnp.float32),
        pltpu.VMEM((CZR, KL), jnp.float32),
        pltpu.SemaphoreType.DMA,
        pltpu.SemaphoreType.DMA,
        pltpu.VMEM_SHARED((N, KL), jnp.float32),
    ],
    compiler_params=_SC_PARAMS,
)
def _count_kernel(dst_hbm, typ_hbm, c_out_hbm,
                  dst_v, typ_v, dkey2, rkey2, vals0, vals1, czbuf,
                  ss0, ss1, c_sh):
    cid = lax.axis_index("c")
    sid = lax.axis_index("s")
    gid = cid * NSUB + sid
    one16 = jnp.ones((LANES,), jnp.float32)
    zero16 = jnp.zeros((LANES,), jnp.float32)
    iota16 = lax.iota(jnp.int32, LANES)

    def fill_zero(i, _):
        czbuf[i, :] = zero16
        return 0
    lax.fori_loop(0, CZR, fill_zero, 0)
    for k in range(CTB // CZR):
        pltpu.sync_copy(czbuf, c_sh.at[pl.ds(sid * CTB + k * CZR, CZR)])

    @pl.when(sid == NSUB - 1)
    def _():
        pltpu.sync_copy(czbuf.at[pl.ds(0, CTT)],
                        c_sh.at[pl.ds(NSUB * CTB, CTT)])
    plsc.subcore_barrier()

    vbufs = (vals0, vals1)
    ssems = (ss0, ss1)

    def build_vals(j, b):
        vals = vbufs[b]
        for v in range(CW // LANES):
            r16 = rkey2[j, pl.ds(v * LANES, LANES)]
            for t in range(LANES):
                rs = r16[t]
                vals[v * LANES + t, :] = jnp.where(iota16 == rs, one16,
                                                   zero16)

    def issue(j, b):
        pltpu.async_copy(vbufs[b], c_sh.at[dkey2.at[j]], ssems[b], add=True)

    def wait(j, b):
        pltpu.make_async_copy(vbufs[b], c_sh.at[dkey2.at[j]],
                              ssems[b]).wait()

    def super_chunk(sc, _):
        eb = pl.multiple_of(gid * ETA + sc * ASC, 8)
        pltpu.sync_copy(dst_hbm.at[pl.ds(eb, ASC)], dst_v)
        pltpu.sync_copy(typ_hbm.at[pl.ds(eb, ASC)], typ_v)

        def build(j, _):
            for v in range(CW // LANES):
                off = j * CW + v * LANES
                sl = pl.ds(v * LANES, LANES)
                dkey2[j, sl] = dst_v[pl.ds(off, LANES)]
                rkey2[j, sl] = typ_v[pl.ds(off, LANES)]
            return 0
        lax.fori_loop(0, ANCH, build, 0)

        def pair(i, _):
            j0 = 2 * i

            @pl.when(i > 0)
            def _():
                wait(j0 - 2, 0)
            build_vals(j0, 0)
            issue(j0, 0)

            @pl.when(i > 0)
            def _():
                wait(j0 - 1, 1)
            build_vals(j0 + 1, 1)
            issue(j0 + 1, 1)
            return 0
        lax.fori_loop(0, ANP, pair, 0)
        wait(ANCH - 3, 0)
        build_vals(ANCH - 1, 0)
        issue(ANCH - 1, 0)
        wait(ANCH - 2, 1)
        wait(ANCH - 1, 0)
        return 0
    lax.fori_loop(0, ANSC, super_chunk, 0)
    plsc.subcore_barrier()

    pltpu.sync_copy(c_sh.at[pl.ds(sid * CTB, CTB)],
                    c_out_hbm.at[pl.ds(cid * N + sid * CTB, CTB)])

    @pl.when(sid == NSUB - 1)
    def _():
        pltpu.sync_copy(c_sh.at[pl.ds(NSUB * CTB, CTT)],
                        c_out_hbm.at[pl.ds(cid * N + NSUB * CTB, CTT)])


ET = E // (NCORES * NSUB)
SCHUNK = 2000
NSC = ET // SCHUNK
NCH_S = SCHUNK // CW
NPAIR = NCH_S // 2
NT = 624
NT_TAIL = N - NSUB * NT
ZR = 48

_mesh_c = plsc.VectorSubcoreMesh(core_axis_name="c", subcore_axis_name="s")


@functools.partial(
    pl.kernel,
    out_type=jax.ShapeDtypeStruct((NCORES * N, H), jnp.float32),
    mesh=_mesh_c,
    scratch_types=[
        pltpu.VMEM((SCHUNK,), jnp.int32),
        pltpu.VMEM((SCHUNK,), jnp.int32),
        pltpu.VMEM((SCHUNK,), jnp.int32),
        pltpu.VMEM((NCH_S, CW), jnp.int32),
        pltpu.VMEM((NCH_S, CW), jnp.int32),
        pltpu.VMEM((NCH_S, CW), jnp.int32),
        pltpu.VMEM((CW,), jnp.float32),
        pltpu.VMEM((CW,), jnp.float32),
        pltpu.VMEM((CW, H), jnp.float32),
        pltpu.VMEM((CW, H), jnp.float32),
        pltpu.VMEM((ZR, H), jnp.float32),
        pltpu.SemaphoreType.DMA,
        pltpu.SemaphoreType.DMA,
        pltpu.SemaphoreType.DMA,
        pltpu.SemaphoreType.DMA,
        pltpu.VMEM_SHARED((N, H), jnp.float32),
    ],
    compiler_params=_SC_PARAMS,
)
def _msg_kernel(xr_hbm, w_hbm, src_hbm, dst_hbm, typ_hbm, out_hbm,
                src_v, dst_v, typ_v, ksrc2, wkey2, dst2,
                wv0, wv1, rows0, rows1, zbuf, gs0, gs1, ws0, ws1, acc_sh):
    cid = lax.axis_index("c")
    sid = lax.axis_index("s")
    gid = cid * NSUB + sid
    zero16 = jnp.zeros((LANES,), jnp.float32)

    def zb(i, _):
        for q in range(H // LANES):
            zbuf[i, pl.ds(q * LANES, LANES)] = zero16
        return 0
    lax.fori_loop(0, ZR, zb, 0)
    for k in range(NT // ZR):
        pltpu.sync_copy(zbuf, acc_sh.at[pl.ds(sid * NT + k * ZR, ZR)])

    @pl.when(sid == NSUB - 1)
    def _():
        pltpu.sync_copy(zbuf.at[pl.ds(0, NT_TAIL)],
                        acc_sh.at[pl.ds(NSUB * NT, NT_TAIL)])
    plsc.subcore_barrier()

    rbufs = (rows0, rows1)
    wbufs = (wv0, wv1)
    gsems = (gs0, gs1)
    wsems = (ws0, ws1)

    def issue(j, b):
        pltpu.async_copy(xr_hbm.at[ksrc2.at[j]], rbufs[b], gsems[b])
        pltpu.async_copy(w_hbm.at[wkey2.at[j]], wbufs[b], wsems[b])

    def wait(j, b):
        pltpu.make_async_copy(xr_hbm.at[ksrc2.at[j]], rbufs[b],
                              gsems[b]).wait()
        pltpu.make_async_copy(w_hbm.at[wkey2.at[j]], wbufs[b],
                              wsems[b]).wait()

    def process(j, b):
        rows, wv = rbufs[b], wbufs[b]
        for v in range(CW // LANES):
            w16 = wv[pl.ds(v * LANES, LANES)]
            for t in range(LANES):
                e = v * LANES + t
                ws = w16[t]
                for q in range(H // LANES):
                    sl = pl.ds(q * LANES, LANES)
                    rows[e, sl] = rows[e, sl] * ws
        pltpu.sync_copy(rows, acc_sh.at[dst2.at[j]], add=True)

    def super_chunk(sc, _):
        eb = pl.multiple_of(gid * ET + sc * SCHUNK, 8)
        pltpu.sync_copy(src_hbm.at[pl.ds(eb, SCHUNK)], src_v)
        pltpu.sync_copy(dst_hbm.at[pl.ds(eb, SCHUNK)], dst_v)
        pltpu.sync_copy(typ_hbm.at[pl.ds(eb, SCHUNK)], typ_v)

        def build(j, _):
            for v in range(CW // LANES):
                off = j * CW + v * LANES
                s16 = src_v[pl.ds(off, LANES)]
                d16 = dst_v[pl.ds(off, LANES)]
                t16 = typ_v[pl.ds(off, LANES)]
                sl = pl.ds(v * LANES, LANES)
                ksrc2[j, sl] = t16 * N + s16
                wkey2[j, sl] = d16 * KL + t16
                dst2[j, sl] = d16
            return 0
        lax.fori_loop(0, NCH_S, build, 0)

        issue(0, 0)

        def pair(i, _):
            j0 = 2 * i
            wait(j0, 0)
            issue(j0 + 1, 1)
            process(j0, 0)
            wait(j0 + 1, 1)
            issue(j0 + 2, 0)
            process(j0 + 1, 1)
            return 0
        lax.fori_loop(0, NPAIR, pair, 0)
        wait(NCH_S - 1, 0)
        process(NCH_S - 1, 0)
        return 0
    lax.fori_loop(0, NSC, super_chunk, 0)
    plsc.subcore_barrier()

    pltpu.sync_copy(acc_sh.at[pl.ds(sid * NT, NT)],
                    out_hbm.at[pl.ds(cid * N + sid * NT, NT)])

    @pl.when(sid == NSUB - 1)
    def _():
        pltpu.sync_copy(acc_sh.at[pl.ds(NSUB * NT, NT_TAIL)],
                        out_hbm.at[pl.ds(cid * N + NSUB * NT, NT_TAIL)])


QT = Q // (NCORES * NSUB)


@functools.partial(
    pl.kernel,
    out_type=jax.ShapeDtypeStruct((Q, H), jnp.float32),
    mesh=_mesh_c,
    scratch_types=[
        pltpu.VMEM((QT,), jnp.int32),
        pltpu.VMEM((QT,), jnp.int32),
        pltpu.VMEM((QT,), jnp.int32),
        pltpu.VMEM((QT, H), jnp.float32),
        pltpu.VMEM((QT, H), jnp.float32),
        pltpu.VMEM((QT, H), jnp.float32),
    ],
    compiler_params=_SC_PARAMS,
)
def _gather_kernel(xr_hbm, parts_hbm, nidx_hbm, out_hbm,
                   i0, i1, i2, b0, b1, b2):
    cid = lax.axis_index("c")
    sid = lax.axis_index("s")
    gid = cid * NSUB + sid
    qb = gid * QT
    pltpu.sync_copy(nidx_hbm.at[pl.ds(qb, QT)], i1)

    def shift(v, _):
        sl = pl.ds(v * LANES, LANES)
        base = i1[sl]
        i0[sl] = base + (R * N)
        i2[sl] = base + N
        return 0
    lax.fori_loop(0, QT // LANES, shift, 0)

    pltpu.sync_copy(xr_hbm.at[i0], b0)
    pltpu.sync_copy(parts_hbm.at[i1], b1)
    pltpu.sync_copy(parts_hbm.at[i2], b2)

    def add(i, _):
        for q in range(H // LANES):
            sl = pl.ds(q * LANES, LANES)
            b0[i, sl] = b0[i, sl] + b1[i, sl] + b2[i, sl]
        return 0
    lax.fori_loop(0, QT, add, 0)
    pltpu.sync_copy(b0, out_hbm.at[pl.ds(qb, QT)])


BN = 1000
NBLK = N // BN


def _xr_block(h, comp_ref, bases_ref, root_ref, out_ref, r):
    @pl.when(r < R)
    def _():
        w = (comp_ref[r, 0] * bases_ref[0] + comp_ref[r, 1] * bases_ref[1]
             + comp_ref[r, 2] * bases_ref[2] + comp_ref[r, 3] * bases_ref[3])
        out_ref[0] = jnp.dot(h, w, preferred_element_type=jnp.float32)

    @pl.when(r == R)
    def _():
        out_ref[0] = jnp.dot(h, root_ref[...],
                             preferred_element_type=jnp.float32)


def _b1_body(comp_ref, x_ref, bases_ref, root_ref, out_ref):
    r = pl.program_id(1)
    _xr_block(x_ref[...], comp_ref, bases_ref, root_ref, out_ref, r)


def _b2_body(comp_ref, s_ref, p0_ref, p1_ref, b1_ref, g_ref, bb_ref,
             bases_ref, root_ref, out_ref):
    r = pl.program_id(1)
    x = s_ref[0] + b1_ref[...] + p0_ref[...] + p1_ref[...]
    mu = jnp.mean(x, axis=-1, keepdims=True)
    var = jnp.mean((x - mu) ** 2, axis=-1, keepdims=True)
    x = (x - mu) * lax.rsqrt(var + 1e-5) * g_ref[...] + bb_ref[...]
    h = jnp.maximum(x, 0.0)
    _xr_block(h, comp_ref, bases_ref, root_ref, out_ref, r)


def _head_body(q_ref, b2_ref, g_ref, bb_ref, w1_ref, c1_ref, w2_ref, c2_ref,
               out_ref):
    x = q_ref[...] + b2_ref[...]
    mu = jnp.mean(x, axis=-1, keepdims=True)
    var = jnp.mean((x - mu) ** 2, axis=-1, keepdims=True)
    x = (x - mu) * lax.rsqrt(var + 1e-5) * g_ref[...] + bb_ref[...]
    h = jnp.maximum(
        jnp.dot(x, w1_ref[...], preferred_element_type=jnp.float32)
        + c1_ref[...], 0.0)
    out_ref[...] = (jnp.dot(h, w2_ref[...], preferred_element_type=jnp.float32)
                    + c2_ref[...])


_vec_spec = pl.BlockSpec((1, H), lambda nb, r: (0, 0))
_b1_call = pl.pallas_call(
    _b1_body,
    grid=(NBLK, R + 1),
    in_specs=[
        pl.BlockSpec(memory_space=pltpu.SMEM),
        pl.BlockSpec((BN, H), lambda nb, r: (nb, 0)),
        pl.BlockSpec((NB_BASES, H, H), lambda nb, r: (0, 0, 0)),
        pl.BlockSpec((H, H), lambda nb, r: (0, 0)),
    ],
    out_specs=pl.BlockSpec((1, BN, H), lambda nb, r: (r, nb, 0)),
    out_shape=jax.ShapeDtypeStruct((R + 1, N, H), jnp.float32),
)

_b2_call = pl.pallas_call(
    _b2_body,
    grid=(NBLK, R + 1),
    in_specs=[
        pl.BlockSpec(memory_space=pltpu.SMEM),
        pl.BlockSpec((1, BN, H), lambda nb, r: (R, nb, 0)),
        pl.BlockSpec((BN, H), lambda nb, r: (nb, 0)),
        pl.BlockSpec((BN, H), lambda nb, r: (NBLK + nb, 0)),
        _vec_spec,
        _vec_spec,
        _vec_spec,
        pl.BlockSpec((NB_BASES, H, H), lambda nb, r: (0, 0, 0)),
        pl.BlockSpec((H, H), lambda nb, r: (0, 0)),
    ],
    out_specs=pl.BlockSpec((1, BN, H), lambda nb, r: (r, nb, 0)),
    out_shape=jax.ShapeDtypeStruct((R + 1, N, H), jnp.float32),
)

_head_call = pl.pallas_call(
    _head_body,
    out_shape=jax.ShapeDtypeStruct((Q, H), jnp.float32),
)


def _wcomb_body(c_ref, w_ref):
    w_ref[...] = 1.0 / jnp.maximum(c_ref[0] + c_ref[1], 1.0)


_wcomb_call = pl.pallas_call(
    _wcomb_body,
    out_shape=jax.ShapeDtypeStruct((N * KL // H, H), jnp.float32),
)


def kernel(edge_index, edge_type, node_indices, emb, w_bases1, comp1, root1,
           bias1, w_bases2, comp2, root2, bias2, ln1_g, ln1_b, ln2_g, ln2_b,
           cls_w1, cls_b1, cls_w2, cls_b2):
    src = edge_index[0].astype(jnp.int32)
    dst = edge_index[1].astype(jnp.int32)
    typ = edge_type.astype(jnp.int32)
    nidx = node_indices.astype(jnp.int32)

    c_parts = _count_kernel(dst, typ)
    w_node = _wcomb_call(c_parts.reshape(2, N * KL // H, H))
    wflat = w_node.reshape(N * KL)
    xr1 = _b1_call(comp1, emb, w_bases1, root1)
    parts1 = _msg_kernel(xr1.reshape((R + 1) * N, H), wflat, src, dst, typ)
    xr2 = _b2_call(comp2, xr1, parts1, parts1,
                   bias1.reshape(1, H), ln1_g.reshape(1, H),
                   ln1_b.reshape(1, H), w_bases2, root2)
    xr2f = xr2.reshape((R + 1) * N, H)
    parts2 = _msg_kernel(xr2f, wflat, src, dst, typ)
    q = _gather_kernel(xr2f, parts2, nidx)

    w2p = jnp.pad(cls_w2, ((0, 0), (0, H - NCLS)))
    b2p = jnp.pad(cls_b2, (0, H - NCLS)).reshape(1, H)
    logits = _head_call(q, bias2.reshape(1, H), ln2_g.reshape(1, H),
                        ln2_b.reshape(1, H), cls_w1, cls_b1.reshape(1, H),
                        w2p, b2p)
    return logits[:, :NCLS]
